# Initial kernel scaffold; baseline (speedup 1.0000x reference)
import jax
import jax.numpy as jnp
from jax.experimental import pallas as pl


def kernel(test_feature, train_features, train_labels):
    raise NotImplementedError("write your pallas kernel here")


# speedup vs baseline: 18.4900x; 18.4900x over previous
import jax
import jax.numpy as jnp
from jax.experimental import pallas as pl

BS = 8
P = 196
D = 384
T = 512
K = 20
NUM_CLASSES = 21
PS = 16
NROWS = 14


def _patch_body(tf_ref, trf_ref, lab_ref, out_ref):
    tf = tf_ref[0]
    trf = trf_ref[0]
    sim = jnp.dot(tf, trf, preferred_element_type=jnp.float32)

    cur = sim
    iota = jax.lax.broadcasted_iota(jnp.int32, (BS, T), 1)
    masks = []
    ms = []
    for _ in range(K):
        m = jnp.max(cur, axis=1, keepdims=True)
        e = cur == m
        first = jnp.min(jnp.where(e, iota, T), axis=1, keepdims=True)
        e = iota == first
        masks.append(e.astype(jnp.float32))
        ms.append(m)
        cur = jnp.where(e, -jnp.inf, cur)
    S = jnp.stack(masks, axis=1)
    mk = jnp.concatenate(ms, axis=1)
    w = jax.nn.softmax(mk, axis=1)

    labf = lab_ref[0].astype(jnp.float32)
    labc = jax.lax.dot_general(
        S.reshape(BS * K, T), labf, (((1,), (1,)), ((), ())),
        preferred_element_type=jnp.float32,
    ).reshape(BS, K, PS * PS)

    best_v = jnp.full((BS, PS * PS), -1.0, jnp.float32)
    best_c = jnp.zeros((BS, PS * PS), jnp.int32)
    for c in range(NUM_CLASSES):
        vc = jnp.sum(jnp.where(labc == float(c), w[:, :, None], 0.0), axis=1)
        upd = vc > best_v
        best_v = jnp.where(upd, vc, best_v)
        best_c = jnp.where(upd, c, best_c)
    out_ref[0] = best_c


def kernel(test_feature, train_features, train_labels):
    tf_t = jnp.transpose(test_feature, (1, 0, 2))
    pred_patch = pl.pallas_call(
        _patch_body,
        grid=(P,),
        in_specs=[
            pl.BlockSpec((1, BS, D), lambda p: (p, 0, 0)),
            pl.BlockSpec((1, D, T), lambda p: (p, 0, 0)),
            pl.BlockSpec((1, PS * PS, T), lambda p: (p, 0, 0)),
        ],
        out_specs=pl.BlockSpec((1, BS, PS * PS), lambda p: (p, 0, 0)),
        out_shape=jax.ShapeDtypeStruct((P, BS, PS * PS), jnp.int32),
    )(tf_t, train_features, train_labels)
    img = jnp.transpose(pred_patch, (1, 0, 2)).reshape(BS, NROWS, NROWS, PS, PS)
    img = jnp.transpose(img, (0, 1, 3, 2, 4)).reshape(BS, NROWS * PS, NROWS * PS)
    return img


# speedup vs baseline: 51.9652x; 2.8104x over previous
import jax
import jax.numpy as jnp
from jax.experimental import pallas as pl

BS = 8
P = 196
D = 384
T = 512
K = 20
NUM_CLASSES = 21
PS = 16
NROWS = 14
G = 7


def _patch_body(tf_ref, trf_ref, lab_ref, out_ref):
    tf = tf_ref[...]
    trf = trf_ref[...]
    sim = jax.lax.dot_general(
        tf, trf, (((2,), (1,)), ((0,), (0,))),
        preferred_element_type=jnp.float32,
    ).reshape(G * BS, T)

    cur = sim
    iota = jax.lax.broadcasted_iota(jnp.int32, (G * BS, T), 1)
    masks = []
    ms = []
    for _ in range(K):
        m = jnp.max(cur, axis=1, keepdims=True)
        e = cur == m
        first = jnp.min(jnp.where(e, iota, T), axis=1, keepdims=True)
        e = iota == first
        masks.append(e.astype(jnp.float32))
        ms.append(m)
        cur = jnp.where(e, -jnp.inf, cur)
    S = jnp.stack(masks, axis=1)
    mk = jnp.concatenate(ms, axis=1)
    w = jax.nn.softmax(mk, axis=1)

    labf = lab_ref[...].astype(jnp.float32)
    labc = jax.lax.dot_general(
        S.reshape(G, BS * K, T), labf, (((2,), (2,)), ((0,), (0,))),
        preferred_element_type=jnp.float32,
    ).reshape(G * BS, K, PS * PS)

    best_v = jnp.full((G * BS, PS * PS), -1.0, jnp.float32)
    best_c = jnp.zeros((G * BS, PS * PS), jnp.int32)
    for c in range(NUM_CLASSES):
        vc = jnp.sum(jnp.where(labc == float(c), w[:, :, None], 0.0), axis=1)
        upd = vc > best_v
        best_v = jnp.where(upd, vc, best_v)
        best_c = jnp.where(upd, c, best_c)
    out_ref[...] = best_c.reshape(G, BS, PS * PS)


def kernel(test_feature, train_features, train_labels):
    tf_t = jnp.transpose(test_feature, (1, 0, 2))
    pred_patch = pl.pallas_call(
        _patch_body,
        grid=(P // G,),
        in_specs=[
            pl.BlockSpec((G, BS, D), lambda p: (p, 0, 0)),
            pl.BlockSpec((G, D, T), lambda p: (p, 0, 0)),
            pl.BlockSpec((G, PS * PS, T), lambda p: (p, 0, 0)),
        ],
        out_specs=pl.BlockSpec((G, BS, PS * PS), lambda p: (p, 0, 0)),
        out_shape=jax.ShapeDtypeStruct((P, BS, PS * PS), jnp.int32),
    )(tf_t, train_features, train_labels)
    img = jnp.transpose(pred_patch, (1, 0, 2)).reshape(BS, NROWS, NROWS, PS, PS)
    img = jnp.transpose(img, (0, 1, 3, 2, 4)).reshape(BS, NROWS * PS, NROWS * PS)
    return img
